# baseline (speedup 1.0000x reference)
import functools

import jax
import jax.numpy as jnp
from jax import lax
from jax.experimental import pallas as pl
from jax.experimental.pallas import tpu as pltpu
from jax.experimental.pallas import tpu_sc as plsc

N_NODES = 10000
N_EDGES = 320000

NC, NS, LANES = 2, 16, 16
NW = NC * NS
CHUNK = 128
EPAD = 327680
NCHUNKS = EPAD // CHUNK
NPAD = 10112
ROWS_PER_SUB = NPAD // NS

_sc_mesh = plsc.VectorSubcoreMesh(
    core_axis_name="c", subcore_axis_name="s", num_cores=NC, num_subcores=NS
)

_untiled = pltpu.CompilerParams(use_tc_tiling_on_sc=False)


def _sub_slice(ref, s):
    return ref.at[pl.ds(s * ROWS_PER_SUB, ROWS_PER_SUB)]


DEG_CH = NCHUNKS // NW

_deg_part = jax.ShapeDtypeStruct((NPAD, LANES), jnp.float32)


@functools.partial(
    pl.kernel,
    out_type=(_deg_part, _deg_part, _deg_part, _deg_part),
    mesh=_sc_mesh,
    compiler_params=_untiled,
    scratch_types=[
        pltpu.VMEM((DEG_CH, CHUNK), jnp.int32),
        pltpu.VMEM((DEG_CH, CHUNK), jnp.int32),
        pltpu.VMEM((CHUNK, LANES), jnp.float32),
        pltpu.VMEM_SHARED((NPAD, LANES), jnp.float32),
        pltpu.VMEM_SHARED((NPAD, LANES), jnp.float32),
        pltpu.SemaphoreType.DMA,
        pltpu.SemaphoreType.DMA,
    ],
)
def _sc_degrees(src_hbm, dst_hbm, ones_hbm, zeros_hbm,
                dego0, dego1, degi0, degi1,
                src_v, dst_v, ones_v, dego_sh, degi_sh, dsem, isem):
    c = lax.axis_index("c")
    s = lax.axis_index("s")
    base = (c * NS + s) * DEG_CH
    pltpu.sync_copy(src_hbm.at[pl.ds(base, DEG_CH)], src_v)
    pltpu.sync_copy(dst_hbm.at[pl.ds(base, DEG_CH)], dst_v)
    pltpu.sync_copy(ones_hbm, ones_v)
    pltpu.sync_copy(_sub_slice(zeros_hbm, s), _sub_slice(dego_sh, s))
    pltpu.sync_copy(_sub_slice(zeros_hbm, s), _sub_slice(degi_sh, s))
    plsc.subcore_barrier()

    @pl.loop(0, DEG_CH)
    def _(j):
        o_cp = pltpu.make_async_copy(ones_v, dego_sh.at[src_v.at[j]], dsem)
        i_cp = pltpu.make_async_copy(ones_v, degi_sh.at[dst_v.at[j]], isem)
        o_cp.start(add=True)
        i_cp.start(add=True)
        o_cp.wait()
        i_cp.wait()

    plsc.subcore_barrier()

    @pl.when(c == 0)
    def _():
        pltpu.sync_copy(_sub_slice(dego_sh, s), _sub_slice(dego0, s))
        pltpu.sync_copy(_sub_slice(degi_sh, s), _sub_slice(degi0, s))

    @pl.when(c == 1)
    def _():
        pltpu.sync_copy(_sub_slice(dego_sh, s), _sub_slice(dego1, s))
        pltpu.sync_copy(_sub_slice(degi_sh, s), _sub_slice(degi1, s))


def _make_sc_edge_pass(feature_split):
    n_ch = NCHUNKS // NS if feature_split else NCHUNKS // NW
    Fh = 64

    @functools.partial(
        pl.kernel,
        out_type=(jax.ShapeDtypeStruct((NPAD, Fh), jnp.float32),
                  jax.ShapeDtypeStruct((NPAD, Fh), jnp.float32)),
        mesh=_sc_mesh,
        compiler_params=_untiled,
        scratch_types=[
            pltpu.VMEM((n_ch, CHUNK), jnp.int32),
            pltpu.VMEM((n_ch, CHUNK), jnp.int32),
            pltpu.VMEM((CHUNK, Fh), jnp.float32),
            pltpu.VMEM((CHUNK, Fh), jnp.float32),
            pltpu.VMEM_SHARED((NPAD, Fh), jnp.float32),
            pltpu.SemaphoreType.DMA,
            pltpu.SemaphoreType.DMA,
            pltpu.SemaphoreType.DMA,
            pltpu.SemaphoreType.DMA,
        ],
    )
    def edge_pass(h_hbm, src_hbm, dst_hbm, zeros_hbm, out0, out1,
                  src_v, dst_v, rows_a, rows_b, agg_sh,
                  gsem_a, gsem_b, ssem_a, ssem_b):
        c = lax.axis_index("c")
        s = lax.axis_index("s")
        if feature_split:
            src_base = c * NCHUNKS + s * n_ch
            dst_base = s * n_ch
        else:
            src_base = dst_base = (c * NS + s) * n_ch
        pltpu.sync_copy(src_hbm.at[pl.ds(src_base, n_ch)], src_v)
        pltpu.sync_copy(dst_hbm.at[pl.ds(dst_base, n_ch)], dst_v)
        pltpu.sync_copy(_sub_slice(zeros_hbm, s), _sub_slice(agg_sh, s))
        plsc.subcore_barrier()

        def gather(j, buf, sem):
            return pltpu.make_async_copy(h_hbm.at[src_v.at[j]], buf, sem)

        def scat(j, buf, sem):
            return pltpu.make_async_copy(buf, agg_sh.at[dst_v.at[j]], sem)

        def stage(j, buf, gsem, ssem):
            gather(j, buf, gsem).wait()
            scat(j, buf, ssem).start(add=True)

        gather(0, rows_a, gsem_a).start()

        @pl.loop(0, n_ch // 2 - 1)
        def _(i):
            j = 2 * i
            stage(j, rows_a, gsem_a, ssem_a)
            gather(j + 1, rows_b, gsem_b).start()
            stage(j + 1, rows_b, gsem_b, ssem_b)
            scat(j, rows_a, ssem_a).wait()
            gather(j + 2, rows_a, gsem_a).start()
            scat(j + 1, rows_b, ssem_b).wait()

        j = n_ch - 2
        stage(j, rows_a, gsem_a, ssem_a)
        gather(j + 1, rows_b, gsem_b).start()
        stage(j + 1, rows_b, gsem_b, ssem_b)
        scat(j, rows_a, ssem_a).wait()
        scat(j + 1, rows_b, ssem_b).wait()

        plsc.subcore_barrier()

        @pl.when(c == 0)
        def _():
            pltpu.sync_copy(_sub_slice(agg_sh, s), _sub_slice(out0, s))

        @pl.when(c == 1)
        def _():
            pltpu.sync_copy(_sub_slice(agg_sh, s), _sub_slice(out1, s))

    return edge_pass


_sc_edge_pass_l1 = _make_sc_edge_pass(True)
_sc_edge_pass_l2 = _make_sc_edge_pass(False)


GRID = 8
BLK = NPAD // GRID


def _rs(d0, d1):
    d = d0[...] + d1[...]
    return lax.rsqrt(jnp.maximum(d[:, :1], 1.0))


def _tc_layer1(x_ref, w_ref, do0_ref, do1_ref, o_ref):
    c = pl.program_id(0)
    h = jnp.dot(x_ref[...], w_ref[...], preferred_element_type=jnp.float32)
    h = h * _rs(do0_ref, do1_ref)
    o_ref[...] = jnp.where(c == 0, h[:, :64], h[:, 64:])


def _tc_mid(a0_ref, a1_ref, di0_ref, di1_ref, do0_ref, do1_ref,
            b1_ref, w_ref, o_ref):
    a = jnp.concatenate([a0_ref[...], a1_ref[...]], axis=1)
    h = jnp.maximum(a * _rs(di0_ref, di1_ref) + b1_ref[...], 0.0)
    o_ref[...] = jnp.dot(h, w_ref[...],
                         preferred_element_type=jnp.float32) * _rs(do0_ref,
                                                                   do1_ref)


def _tc_final(p0_ref, p1_ref, di0_ref, di1_ref, b2_ref, o_ref):
    a = p0_ref[...] + p1_ref[...]
    o_ref[...] = a * _rs(di0_ref, di1_ref) + b2_ref[...]


def _blk(width, index_map=lambda i: (i, 0)):
    return pl.BlockSpec((BLK, width), index_map)


def _full_spec(r, cw):
    return pl.BlockSpec((r, cw), lambda *_: (0, 0))


def kernel(x, edge_index, W1, b1, W2, b2):
    f32 = jnp.float32
    src = edge_index[0].astype(jnp.int32)
    dst = edge_index[1].astype(jnp.int32)
    pad = N_NODES + (jnp.arange(EPAD - N_EDGES, dtype=jnp.int32)
                     % (NPAD - N_NODES))
    src2d = jnp.concatenate([src, pad]).reshape(NCHUNKS, CHUNK)
    dst2d = jnp.concatenate([dst, pad]).reshape(NCHUNKS, CHUNK)
    srcadj = jnp.concatenate([src2d, src2d + NPAD])

    ones16 = jnp.ones((CHUNK, LANES), f32)
    zeros16 = jnp.zeros((NPAD, LANES), f32)
    zeros64 = jnp.zeros((NPAD, 64), f32)

    do0, do1, di0, di1 = _sc_degrees(src2d, dst2d, ones16, zeros16)

    h1 = pl.pallas_call(
        _tc_layer1,
        grid=(NC, GRID),
        in_specs=[pl.BlockSpec((BLK, 128), lambda c, i: (i, 0)),
                  _full_spec(128, 128),
                  pl.BlockSpec((BLK, LANES), lambda c, i: (i, 0)),
                  pl.BlockSpec((BLK, LANES), lambda c, i: (i, 0))],
        out_specs=pl.BlockSpec((BLK, 64), lambda c, i: (c * GRID + i, 0)),
        out_shape=jax.ShapeDtypeStruct((NC * NPAD, 64), f32),
    )(x, W1, do0, do1)

    a0, a1 = _sc_edge_pass_l1(h1, srcadj, dst2d, zeros64)

    h2 = pl.pallas_call(
        _tc_mid,
        grid=(GRID,),
        in_specs=[_blk(64), _blk(64), _blk(LANES), _blk(LANES),
                  _blk(LANES), _blk(LANES),
                  _full_spec(1, 128), _full_spec(128, 64)],
        out_specs=_blk(64),
        out_shape=jax.ShapeDtypeStruct((NPAD, 64), f32),
    )(a0, a1, di0, di1, do0, do1, b1.reshape(1, 128), W2)

    p0, p1 = _sc_edge_pass_l2(h2, src2d, dst2d, zeros64)

    FBLK = 1000
    fblk = lambda w: pl.BlockSpec((FBLK, w), lambda i: (i, 0))
    out = pl.pallas_call(
        _tc_final,
        grid=(N_NODES // FBLK,),
        in_specs=[fblk(64), fblk(64), fblk(LANES), fblk(LANES),
                  _full_spec(1, 64)],
        out_specs=fblk(64),
        out_shape=jax.ShapeDtypeStruct((N_NODES, 64), f32),
    )(p0, p1, di0, di1, b2.reshape(1, 64))

    return out


# speedup vs baseline: 1.1679x; 1.1679x over previous
import functools

import jax
import jax.numpy as jnp
from jax import lax
from jax.experimental import pallas as pl
from jax.experimental.pallas import tpu as pltpu
from jax.experimental.pallas import tpu_sc as plsc

N_NODES = 10000
N_EDGES = 320000

NC, NS, LANES = 2, 16, 16
NW = NC * NS
CHUNK = 128
EPAD = 327680
NCHUNKS = EPAD // CHUNK
NPAD = 10112
ROWS_PER_SUB = NPAD // NS

_sc_mesh = plsc.VectorSubcoreMesh(
    core_axis_name="c", subcore_axis_name="s", num_cores=NC, num_subcores=NS
)

_untiled = pltpu.CompilerParams(use_tc_tiling_on_sc=False)


def _sub_slice(ref, s):
    return ref.at[pl.ds(s * ROWS_PER_SUB, ROWS_PER_SUB)]


DEG_CH = NCHUNKS // NW

_deg_part = jax.ShapeDtypeStruct((NPAD, LANES), jnp.float32)


@functools.partial(
    pl.kernel,
    out_type=(_deg_part, _deg_part, _deg_part, _deg_part),
    mesh=_sc_mesh,
    compiler_params=_untiled,
    scratch_types=[
        pltpu.VMEM((DEG_CH, CHUNK), jnp.int32),
        pltpu.VMEM((DEG_CH, CHUNK), jnp.int32),
        pltpu.VMEM((CHUNK, LANES), jnp.float32),
        pltpu.VMEM_SHARED((NPAD, LANES), jnp.float32),
        pltpu.VMEM_SHARED((NPAD, LANES), jnp.float32),
        pltpu.SemaphoreType.DMA,
        pltpu.SemaphoreType.DMA,
    ],
)
def _sc_degrees(src_hbm, dst_hbm, ones_hbm, zeros_hbm,
                dego0, dego1, degi0, degi1,
                src_v, dst_v, ones_v, dego_sh, degi_sh, dsem, isem):
    c = lax.axis_index("c")
    s = lax.axis_index("s")
    base = (c * NS + s) * DEG_CH
    pltpu.sync_copy(src_hbm.at[pl.ds(base, DEG_CH)], src_v)
    pltpu.sync_copy(dst_hbm.at[pl.ds(base, DEG_CH)], dst_v)
    pltpu.sync_copy(ones_hbm, ones_v)
    pltpu.sync_copy(_sub_slice(zeros_hbm, s), _sub_slice(dego_sh, s))
    pltpu.sync_copy(_sub_slice(zeros_hbm, s), _sub_slice(degi_sh, s))
    plsc.subcore_barrier()

    @pl.loop(0, DEG_CH)
    def _(j):
        o_cp = pltpu.make_async_copy(ones_v, dego_sh.at[src_v.at[j]], dsem)
        i_cp = pltpu.make_async_copy(ones_v, degi_sh.at[dst_v.at[j]], isem)
        o_cp.start(add=True)
        i_cp.start(add=True)
        o_cp.wait()
        i_cp.wait()

    plsc.subcore_barrier()

    @pl.when(c == 0)
    def _():
        pltpu.sync_copy(_sub_slice(dego_sh, s), _sub_slice(dego0, s))
        pltpu.sync_copy(_sub_slice(degi_sh, s), _sub_slice(degi0, s))

    @pl.when(c == 1)
    def _():
        pltpu.sync_copy(_sub_slice(dego_sh, s), _sub_slice(dego1, s))
        pltpu.sync_copy(_sub_slice(degi_sh, s), _sub_slice(degi1, s))


def _make_sc_edge_pass(feature_split):
    n_ch = NCHUNKS // NS if feature_split else NCHUNKS // NW
    Fh = 64

    @functools.partial(
        pl.kernel,
        out_type=(jax.ShapeDtypeStruct((NPAD, Fh), jnp.float32),
                  jax.ShapeDtypeStruct((NPAD, Fh), jnp.float32)),
        mesh=_sc_mesh,
        compiler_params=_untiled,
        scratch_types=[
            pltpu.VMEM((n_ch, CHUNK), jnp.int32),
            pltpu.VMEM((n_ch, CHUNK), jnp.int32),
            pltpu.VMEM((CHUNK, Fh), jnp.float32),
            pltpu.VMEM((CHUNK, Fh), jnp.float32),
            pltpu.VMEM_SHARED((NPAD, Fh), jnp.float32),
            pltpu.SemaphoreType.DMA,
            pltpu.SemaphoreType.DMA,
            pltpu.SemaphoreType.DMA,
            pltpu.SemaphoreType.DMA,
        ],
    )
    def edge_pass(h_hbm, src_hbm, dst_hbm, zeros_hbm, out0, out1,
                  src_v, dst_v, rows_a, rows_b, agg_sh,
                  gsem_a, gsem_b, ssem_a, ssem_b):
        c = lax.axis_index("c")
        s = lax.axis_index("s")
        if feature_split:
            src_base = c * NCHUNKS + s * n_ch
            dst_base = s * n_ch
        else:
            src_base = dst_base = (c * NS + s) * n_ch
        pltpu.sync_copy(src_hbm.at[pl.ds(src_base, n_ch)], src_v)
        pltpu.sync_copy(dst_hbm.at[pl.ds(dst_base, n_ch)], dst_v)
        pltpu.sync_copy(_sub_slice(zeros_hbm, s), _sub_slice(agg_sh, s))
        plsc.subcore_barrier()

        def gather(j, buf, sem):
            return pltpu.make_async_copy(h_hbm.at[src_v.at[j]], buf, sem)

        def scat(j, buf, sem):
            return pltpu.make_async_copy(buf, agg_sh.at[dst_v.at[j]], sem)

        def stage(j, buf, gsem, ssem):
            gather(j, buf, gsem).wait()
            scat(j, buf, ssem).start(add=True)

        gather(0, rows_a, gsem_a).start()

        @pl.loop(0, n_ch // 2 - 1)
        def _(i):
            j = 2 * i
            stage(j, rows_a, gsem_a, ssem_a)
            gather(j + 1, rows_b, gsem_b).start()
            stage(j + 1, rows_b, gsem_b, ssem_b)
            scat(j, rows_a, ssem_a).wait()
            gather(j + 2, rows_a, gsem_a).start()
            scat(j + 1, rows_b, ssem_b).wait()

        j = n_ch - 2
        stage(j, rows_a, gsem_a, ssem_a)
        gather(j + 1, rows_b, gsem_b).start()
        stage(j + 1, rows_b, gsem_b, ssem_b)
        scat(j, rows_a, ssem_a).wait()
        scat(j + 1, rows_b, ssem_b).wait()

        plsc.subcore_barrier()

        @pl.when(c == 0)
        def _():
            pltpu.sync_copy(_sub_slice(agg_sh, s), _sub_slice(out0, s))

        @pl.when(c == 1)
        def _():
            pltpu.sync_copy(_sub_slice(agg_sh, s), _sub_slice(out1, s))

    return edge_pass


_sc_edge_pass_l2 = _make_sc_edge_pass(False)


L1_NCH = NCHUNKS // NW
L1_NB = 10
L1_BCH = L1_NCH // L1_NB


@functools.partial(
    pl.kernel,
    out_type=(jax.ShapeDtypeStruct((NPAD, 128), jnp.float32),
              jax.ShapeDtypeStruct((NPAD, 128), jnp.float32)),
    mesh=_sc_mesh,
    scratch_types=[
        pltpu.VMEM((2, L1_BCH, CHUNK), jnp.int32),
        pltpu.VMEM((2, L1_BCH, CHUNK), jnp.int32),
        pltpu.VMEM((CHUNK, 128), jnp.float32),
        pltpu.VMEM((CHUNK, 128), jnp.float32),
        pltpu.VMEM_SHARED((NPAD, 128), jnp.float32),
        pltpu.SemaphoreType.DMA,
        pltpu.SemaphoreType.DMA,
        pltpu.SemaphoreType.DMA,
        pltpu.SemaphoreType.DMA,
        pltpu.SemaphoreType.DMA,
    ],
)
def _sc_edge_pass_l1(h_hbm, src_hbm, dst_hbm, zeros_hbm, out0, out1,
                     src_v, dst_v, rows_a, rows_b, agg_sh,
                     gsem_a, gsem_b, ssem_a, ssem_b, xsem):
    c = lax.axis_index("c")
    s = lax.axis_index("s")
    base = (c * NS + s) * L1_NCH

    def idx_cp(b, slot):
        return (
            pltpu.make_async_copy(
                src_hbm.at[pl.ds(base + b * L1_BCH, L1_BCH)],
                src_v.at[slot], xsem),
            pltpu.make_async_copy(
                dst_hbm.at[pl.ds(base + b * L1_BCH, L1_BCH)],
                dst_v.at[slot], xsem),
        )

    def gather(slot, k, buf, sem):
        return pltpu.make_async_copy(h_hbm.at[src_v.at[slot, k]], buf, sem)

    def scat(slot, k, buf, sem):
        return pltpu.make_async_copy(buf, agg_sh.at[dst_v.at[slot, k]], sem)

    a, b0 = idx_cp(0, 0)
    a.start()
    b0.start()
    a.wait()
    b0.wait()
    pltpu.sync_copy(_sub_slice(zeros_hbm, s), _sub_slice(agg_sh, s))
    plsc.subcore_barrier()
    for cp in idx_cp(1, 1):
        cp.start()
    gather(0, 0, rows_a, gsem_a).start()

    def pair(slot, k, lookahead):
        gather(slot, k, rows_a, gsem_a).wait()
        scat(slot, k, rows_a, ssem_a).start(add=True)
        gather(slot, k + 1, rows_b, gsem_b).start()
        gather(slot, k + 1, rows_b, gsem_b).wait()
        scat(slot, k + 1, rows_b, ssem_b).start(add=True)
        scat(slot, k, rows_a, ssem_a).wait()
        if lookahead:
            gather(slot, k + 2, rows_a, gsem_a).start()
        scat(slot, k + 1, rows_b, ssem_b).wait()

    @pl.loop(0, L1_NB)
    def _(b):
        slot = lax.rem(b, 2)
        nslot = lax.rem(b + 1, 2)

        @pl.loop(0, L1_BCH // 2 - 1)
        def _(i):
            pair(slot, 2 * i, True)

        pair(slot, L1_BCH - 2, False)

        @pl.when(b < L1_NB - 1)
        def _():
            for cp in idx_cp(b + 1, nslot):
                cp.wait()
            gather(nslot, 0, rows_a, gsem_a).start()

            @pl.when(b < L1_NB - 2)
            def _():
                for cp in idx_cp(b + 2, slot):
                    cp.start()

    plsc.subcore_barrier()

    @pl.when(c == 0)
    def _():
        pltpu.sync_copy(_sub_slice(agg_sh, s), _sub_slice(out0, s))

    @pl.when(c == 1)
    def _():
        pltpu.sync_copy(_sub_slice(agg_sh, s), _sub_slice(out1, s))


GRID = 8
BLK = NPAD // GRID


def _rs(d0, d1):
    d = d0[...] + d1[...]
    return lax.rsqrt(jnp.maximum(d[:, :1], 1.0))


def _tc_layer1(x_ref, w_ref, do0_ref, do1_ref, o_ref):
    h = jnp.dot(x_ref[...], w_ref[...], preferred_element_type=jnp.float32)
    o_ref[...] = h * _rs(do0_ref, do1_ref)


def _tc_mid(a0_ref, a1_ref, di0_ref, di1_ref, do0_ref, do1_ref,
            b1_ref, w_ref, o_ref):
    a = a0_ref[...] + a1_ref[...]
    h = jnp.maximum(a * _rs(di0_ref, di1_ref) + b1_ref[...], 0.0)
    o_ref[...] = jnp.dot(h, w_ref[...],
                         preferred_element_type=jnp.float32) * _rs(do0_ref,
                                                                   do1_ref)


def _tc_final(p0_ref, p1_ref, di0_ref, di1_ref, b2_ref, o_ref):
    a = p0_ref[...] + p1_ref[...]
    o_ref[...] = a * _rs(di0_ref, di1_ref) + b2_ref[...]


def _blk(width, index_map=lambda i: (i, 0)):
    return pl.BlockSpec((BLK, width), index_map)


def _full_spec(r, cw):
    return pl.BlockSpec((r, cw), lambda *_: (0, 0))


def kernel(x, edge_index, W1, b1, W2, b2):
    f32 = jnp.float32
    src = edge_index[0].astype(jnp.int32)
    dst = edge_index[1].astype(jnp.int32)
    pad = N_NODES + (jnp.arange(EPAD - N_EDGES, dtype=jnp.int32)
                     % (NPAD - N_NODES))
    src2d = jnp.concatenate([src, pad]).reshape(NCHUNKS, CHUNK)
    dst2d = jnp.concatenate([dst, pad]).reshape(NCHUNKS, CHUNK)
    ones16 = jnp.ones((CHUNK, LANES), f32)
    zeros16 = jnp.zeros((NPAD, LANES), f32)
    zeros64 = jnp.zeros((NPAD, 64), f32)
    zeros128 = jnp.zeros((NPAD, 128), f32)

    do0, do1, di0, di1 = _sc_degrees(src2d, dst2d, ones16, zeros16)

    h1 = pl.pallas_call(
        _tc_layer1,
        grid=(GRID,),
        in_specs=[_blk(128), _full_spec(128, 128), _blk(LANES), _blk(LANES)],
        out_specs=_blk(128),
        out_shape=jax.ShapeDtypeStruct((NPAD, 128), f32),
    )(x, W1, do0, do1)

    a0, a1 = _sc_edge_pass_l1(h1, src2d, dst2d, zeros128)

    h2 = pl.pallas_call(
        _tc_mid,
        grid=(GRID,),
        in_specs=[_blk(128), _blk(128), _blk(LANES), _blk(LANES),
                  _blk(LANES), _blk(LANES),
                  _full_spec(1, 128), _full_spec(128, 64)],
        out_specs=_blk(64),
        out_shape=jax.ShapeDtypeStruct((NPAD, 64), f32),
    )(a0, a1, di0, di1, do0, do1, b1.reshape(1, 128), W2)

    p0, p1 = _sc_edge_pass_l2(h2, src2d, dst2d, zeros64)

    FBLK = 1000
    fblk = lambda w: pl.BlockSpec((FBLK, w), lambda i: (i, 0))
    out = pl.pallas_call(
        _tc_final,
        grid=(N_NODES // FBLK,),
        in_specs=[fblk(64), fblk(64), fblk(LANES), fblk(LANES),
                  _full_spec(1, 64)],
        out_specs=fblk(64),
        out_shape=jax.ShapeDtypeStruct((N_NODES, 64), f32),
    )(p0, p1, di0, di1, b2.reshape(1, 64))

    return out
